# Initial kernel scaffold; baseline (speedup 1.0000x reference)
import jax
import jax.numpy as jnp
from jax.experimental import pallas as pl


def kernel(event_type, action, actors, themes, constraints, event_type_emb, action_emb, actor_emb, theme_emb, constraint_emb, W, b):
    raise NotImplementedError("write your pallas kernel here")


# speedup vs baseline: 4.9293x; 4.9293x over previous
import functools

import jax
import jax.numpy as jnp
from jax import lax
from jax.experimental import pallas as pl
from jax.experimental.pallas import tpu as pltpu
from jax.experimental.pallas import tpu_sc as plsc

B, T, K, D = 1024, 50, 8, 128
BT = B * T
L = 16
C = 64


def _proj_body(a_ref, w_ref, b_ref, o_ref, *, zero_first: bool, block_rows: int):
    a = a_ref[...]
    if zero_first:
        row = lax.broadcasted_iota(jnp.int32, a.shape, 0) + pl.program_id(0) * block_rows
        a = jnp.where(row == 0, 0.0, a)
    o_ref[...] = jnp.dot(a, w_ref[...], preferred_element_type=jnp.float32) + b_ref[...]


def _project(table, wblk, bias, zero_first):
    n = table.shape[0]
    r = 1000 if n % 1000 == 0 else n
    grid = n // r
    return pl.pallas_call(
        functools.partial(_proj_body, zero_first=zero_first, block_rows=r),
        grid=(grid,),
        in_specs=[
            pl.BlockSpec((r, D), lambda i: (i, 0)),
            pl.BlockSpec((D, D), lambda i: (0, 0)),
            pl.BlockSpec((1, D), lambda i: (0, 0)),
        ],
        out_specs=pl.BlockSpec((r, D), lambda i: (i, 0)),
        out_shape=jax.ShapeDtypeStruct((n, D), jnp.float32),
    )(table, wblk, bias)


def _make_sc_encode(nc, ns):
    nw = nc * ns
    cpw = BT // nw
    nchunk = cpw // C

    mesh = plsc.VectorSubcoreMesh(core_axis_name="c", subcore_axis_name="s")

    @functools.partial(
        pl.kernel,
        mesh=mesh,
        out_type=jax.ShapeDtypeStruct((BT, D), jnp.float32),
        scratch_types=[
            pltpu.VMEM((C,), jnp.int32),
            pltpu.VMEM((C,), jnp.int32),
            pltpu.VMEM((K, C), jnp.int32),
            pltpu.VMEM((C, D), jnp.float32),
            pltpu.VMEM((C, D), jnp.float32),
            pltpu.VMEM((K, C, D), jnp.float32),
            pltpu.VMEM((C + L,), jnp.float32),
            pltpu.VMEM((C, D), jnp.float32),
            pltpu.SemaphoreType.DMA,
        ],
    )
    def sc_encode(pet, pac, pa, pth, pco, ev, ax, a_t, t_t, c_t, out_hbm,
                  evi, axi, sidx, etr, acr, srows, inv, ob, sem):
        wid = lax.axis_index("s") * nc + lax.axis_index("c")

        def chunk(g, carry):
            base = pl.multiple_of(wid * cpw + g * C, C)
            pltpu.sync_copy(ev.at[pl.ds(base, C)], evi)
            pltpu.sync_copy(ax.at[pl.ds(base, C)], axi)
            cp1 = pltpu.async_copy(pet.at[evi], etr, sem)
            cp2 = pltpu.async_copy(pac.at[axi], acr, sem)
            cp1.wait()
            cp2.wait()

            def init_tok(t, c):
                for d in range(D // L):
                    sl = pl.ds(d * L, L)
                    ob[t, sl] = etr[t, sl] + acr[t, sl]
                return c

            lax.fori_loop(0, C, init_tok, 0)

            for idx_t, tab in ((a_t, pa), (t_t, pth), (c_t, pco)):
                for k in range(K):
                    pltpu.sync_copy(idx_t.at[k, pl.ds(base, C)], sidx.at[k])
                cps = [pltpu.async_copy(tab.at[sidx.at[k]], srows.at[k], sem)
                       for k in range(K)]
                for tg in range(C // L):
                    sl = pl.ds(tg * L, L)
                    cnt = jnp.zeros((L,), jnp.float32)
                    for k in range(K):
                        cnt = cnt + jnp.where(sidx[k, sl] != 0, 1.0, 0.0)
                    inv[sl] = 1.0 / jnp.maximum(cnt, 1.0)
                for cp in cps:
                    cp.wait()

                def tok(t, c):
                    iv = inv[pl.ds(t, L)]
                    ib = jnp.broadcast_to(iv[0], (L,))
                    for d in range(D // L):
                        sl = pl.ds(d * L, L)
                        acc = srows[0, t, sl]
                        for k in range(1, K):
                            acc = acc + srows[k, t, sl]
                        plsc.addupdate(ob.at[t, sl], acc * ib)
                    return c

                lax.fori_loop(0, C, tok, 0)

            pltpu.sync_copy(ob, out_hbm.at[pl.ds(base, C)])
            return carry

        lax.fori_loop(0, nchunk, chunk, 0)

    return sc_encode


def kernel(event_type, action, actors, themes, constraints,
           event_type_emb, action_emb, actor_emb, theme_emb, constraint_emb,
           W, b):
    wr = W.reshape(5, D, D)
    zero_bias = jnp.zeros((1, D), jnp.float32)
    pet = _project(event_type_emb, wr[0], b.reshape(1, D), False)
    pac = _project(action_emb, wr[1], zero_bias, False)
    pa = _project(actor_emb, wr[2], zero_bias, True)
    pth = _project(theme_emb, wr[3], zero_bias, True)
    pco = _project(constraint_emb, wr[4], zero_bias, True)

    ev = event_type.reshape(BT)
    ax = action.reshape(BT)
    a_t = actors.reshape(BT, K).T
    t_t = themes.reshape(BT, K).T
    c_t = constraints.reshape(BT, K).T

    info = plsc.get_sparse_core_info()
    sc_encode = _make_sc_encode(info.num_cores, info.num_subcores)
    out = sc_encode(pet, pac, pa, pth, pco, ev, ax, a_t, t_t, c_t)
    return out.reshape(B, T, D)
